# Initial kernel scaffold; baseline (speedup 1.0000x reference)
import jax
import jax.numpy as jnp
from jax.experimental import pallas as pl


def kernel(data0):
    raise NotImplementedError("write your pallas kernel here")


# speedup vs baseline: 3.6959x; 3.6959x over previous
import functools

import jax
import jax.numpy as jnp
from jax import lax
from jax.experimental import pallas as pl
from jax.experimental.pallas import tpu as pltpu
from jax.experimental.pallas import tpu_sc as plsc

_NF = 2048
_OUT = 64
_ROWS_PER_TILE = 64
_HW = 128


def _sc_mask_body(hands_hbm, mask_hbm, buf, mbuf):
    wid = lax.axis_index("s") * 2 + lax.axis_index("c")
    pltpu.sync_copy(
        hands_hbm.at[pl.ds(wid * (_ROWS_PER_TILE * _HW), _ROWS_PER_TILE * _HW)],
        buf,
    )
    lane = lax.iota(jnp.int32, 16)
    for g in range(_ROWS_PER_TILE // 16):
        row = g * 16 + lane
        acc = jnp.zeros((16,), jnp.float32)
        for col in range(126):
            acc = acc + plsc.load_gather(buf, [row * _HW + col])
        mbuf[pl.ds(g * 16, 16)] = (acc > 0.0).astype(jnp.int32)
    pltpu.sync_copy(mbuf, mask_hbm.at[pl.ds(wid * _ROWS_PER_TILE, _ROWS_PER_TILE)])


@functools.cache
def _sc_mask():
    return pl.kernel(
        _sc_mask_body,
        out_type=jax.ShapeDtypeStruct((_NF,), jnp.int32),
        mesh=plsc.VectorSubcoreMesh(core_axis_name="c", subcore_axis_name="s"),
        scratch_types=[
            pltpu.VMEM((_ROWS_PER_TILE * _HW,), jnp.float32),
            pltpu.VMEM((_ROWS_PER_TILE,), jnp.int32),
        ],
        compiler_params=pltpu.CompilerParams(needs_layout_passes=False),
    )


def _tc_pool_body(lips_ref, rest_ref, mask_ref, out_a_ref, out_b_ref, out_i_ref):
    mf = (mask_ref[...] > 0).astype(jnp.float32)
    p = mf
    d = 1
    while d < _NF:
        shifted = jnp.concatenate(
            [jnp.zeros((1, d), jnp.float32), p[:, : _NF - d]], axis=1
        )
        p = p + shifted
        d *= 2
    K = jnp.sum(mf)
    k = p - 1.0
    L = jnp.where(k <= 0.0, 0.0, 2.0 * k + 32.0)
    U = jnp.where(k >= 2047.0, 4159.0, 2.0 * k + 33.0)
    ovec = lax.broadcasted_iota(jnp.int32, (_OUT, 1), 0).astype(jnp.float32) * 65.0
    lo = jnp.maximum(ovec, L)
    hi = jnp.minimum(ovec + 64.0, U)
    c = jnp.maximum(hi - lo + 1.0, 0.0)
    wgm = c * mf * (1.0 / 65.0)
    lk = jnp.where(K == 0.0, 0.0, jnp.where(K >= 2048.0, 4160.0, 2.0 * K + 32.0))
    t = jnp.maximum((ovec + 64.0) - jnp.maximum(ovec, lk) + 1.0, 0.0) * (1.0 / 65.0)
    dla = lips_ref[...]
    dlb = rest_ref[...]
    out_a_ref[...] = (
        jnp.dot(wgm, dla, preferred_element_type=jnp.float32) + t * dla[0:1, :]
    )
    out_b_ref[...] = (
        jnp.dot(wgm, dlb, preferred_element_type=jnp.float32) + t * dlb[0:1, :]
    )
    jv = lax.broadcasted_iota(jnp.int32, (1, _NF), 1).astype(jnp.float32)
    out_i_ref[...] = jnp.sum(wgm * jv, axis=1, keepdims=True)


_tc_pool = pl.pallas_call(
    _tc_pool_body,
    out_shape=[
        jax.ShapeDtypeStruct((_OUT, 120), jnp.float32),
        jax.ShapeDtypeStruct((_OUT, 237), jnp.float32),
        jax.ShapeDtypeStruct((_OUT, 1), jnp.float32),
    ],
)


def kernel(data0):
    lips = data0[:, 0:40, :].reshape(_NF, 120)
    rest = data0[:, 464:543, :].reshape(_NF, 237)
    hands = jnp.concatenate(
        [
            data0[:, 468:489, :].reshape(_NF, 63),
            data0[:, 522:543, :].reshape(_NF, 63),
            jnp.zeros((_NF, 2), jnp.float32),
        ],
        axis=1,
    )
    mask = _sc_mask()(hands.reshape(-1))
    out_a, out_b, out_i = _tc_pool(lips, rest, mask.reshape(1, _NF))
    data_out = jnp.concatenate([out_a, out_b[:, 12:237]], axis=1)
    return (data_out.reshape(_OUT, 115, 3), out_i.reshape(_OUT))


# speedup vs baseline: 3.7917x; 1.0259x over previous
import functools

import jax
import jax.numpy as jnp
from jax import lax
from jax.experimental import pallas as pl
from jax.experimental.pallas import tpu as pltpu
from jax.experimental.pallas import tpu_sc as plsc

_NF = 2048
_OUT = 64
_ROWS_PER_TILE = 64
_HW = 128


def _sc_mask_body(hands_hbm, mask_hbm, buf, mbuf):
    wid = lax.axis_index("s") * 2 + lax.axis_index("c")
    pltpu.sync_copy(
        hands_hbm.at[pl.ds(wid * (_ROWS_PER_TILE * _HW), _ROWS_PER_TILE * _HW)],
        buf,
    )
    lane = lax.iota(jnp.int32, 16)
    for g in range(_ROWS_PER_TILE // 16):

        def row_step(i, flags):
            base = (g * 16 + i) * _HW
            acc = plsc.load_gather(buf, [lane + base])
            for c0 in (16, 32, 48, 64, 80, 96, 110):
                acc = jnp.maximum(acc, plsc.load_gather(buf, [lane + (base + c0)]))
            hit = (lane == i) & (jnp.max(acc) > 0.0)
            return jnp.where(hit, 1, flags)

        flags = lax.fori_loop(0, 16, row_step, jnp.zeros((16,), jnp.int32))
        mbuf[pl.ds(g * 16, 16)] = flags
    pltpu.sync_copy(mbuf, mask_hbm.at[pl.ds(wid * _ROWS_PER_TILE, _ROWS_PER_TILE)])


@functools.cache
def _sc_mask():
    return pl.kernel(
        _sc_mask_body,
        out_type=jax.ShapeDtypeStruct((_NF,), jnp.int32),
        mesh=plsc.VectorSubcoreMesh(core_axis_name="c", subcore_axis_name="s"),
        scratch_types=[
            pltpu.VMEM((_ROWS_PER_TILE * _HW,), jnp.float32),
            pltpu.VMEM((_ROWS_PER_TILE,), jnp.int32),
        ],
        compiler_params=pltpu.CompilerParams(needs_layout_passes=False),
    )


def _tc_pool_body(lips_ref, rest_ref, mask_ref, out_ref, out_i_ref):
    mf = (mask_ref[...] > 0).astype(jnp.float32)
    p = mf
    d = 1
    while d < _NF:
        shifted = jnp.concatenate(
            [jnp.zeros((1, d), jnp.float32), p[:, : _NF - d]], axis=1
        )
        p = p + shifted
        d *= 2
    K = jnp.sum(mf)
    k = p - 1.0
    L = jnp.where(k <= 0.0, 0.0, 2.0 * k + 32.0)
    U = jnp.where(k >= 2047.0, 4159.0, 2.0 * k + 33.0)
    ovec = lax.broadcasted_iota(jnp.int32, (_OUT, 1), 0).astype(jnp.float32) * 65.0
    lo = jnp.maximum(ovec, L)
    hi = jnp.minimum(ovec + 64.0, U)
    c = jnp.maximum(hi - lo + 1.0, 0.0)
    wgm = c * mf * (1.0 / 65.0)
    lk = jnp.where(K == 0.0, 0.0, jnp.where(K >= 2048.0, 4160.0, 2.0 * K + 32.0))
    t = jnp.maximum((ovec + 64.0) - jnp.maximum(ovec, lk) + 1.0, 0.0) * (1.0 / 65.0)
    dla = lips_ref[...]
    dlb = rest_ref[...]
    out_ref[:, 0:120] = (
        jnp.dot(wgm, dla, preferred_element_type=jnp.float32) + t * dla[0:1, :]
    )
    out_b = jnp.dot(wgm, dlb, preferred_element_type=jnp.float32) + t * dlb[0:1, :]
    out_ref[:, 120:345] = out_b[:, 12:237]
    jv = lax.broadcasted_iota(jnp.int32, (1, _NF), 1).astype(jnp.float32)
    out_i_ref[...] = jnp.sum(wgm * jv, axis=1, keepdims=True)


_tc_pool = pl.pallas_call(
    _tc_pool_body,
    out_shape=[
        jax.ShapeDtypeStruct((_OUT, 345), jnp.float32),
        jax.ShapeDtypeStruct((_OUT, 1), jnp.float32),
    ],
)


def kernel(data0):
    lips = data0[:, 0:40, :].reshape(_NF, 120)
    rest = data0[:, 464:543, :].reshape(_NF, 237)
    hands = jnp.concatenate(
        [
            data0[:, 468:489, :].reshape(_NF, 63),
            data0[:, 522:543, :].reshape(_NF, 63),
            jnp.zeros((_NF, 2), jnp.float32),
        ],
        axis=1,
    )
    mask = _sc_mask()(hands.reshape(-1))
    data_out, out_i = _tc_pool(lips, rest, mask.reshape(1, _NF))
    return (data_out.reshape(_OUT, 115, 3), out_i.reshape(_OUT))
